# baseline (speedup 1.0000x reference)
import jax
import jax.numpy as jnp
from jax import lax
from jax.experimental import pallas as pl
from jax.experimental.pallas import tpu as pltpu
from jax.experimental.pallas import tpu_sc as plsc

N_NODES = 10000
N_EDGES = 320000
D_IN = 128
D_HID = 32
W1 = 40
W2 = 32
COFF = 64

NC, NS = 2, 16
NW = NC * NS
CH = 128
NCHUNK = N_EDGES // CH
TPW = NCHUNK // NW
NEXTRA = NCHUNK - TPW * NW
NBUF = 13
NP = 10240
RPS = NP // NS
BM = 10000


def _proj_body(x_ref, wl_ref, wr_ref, c_ref, b_ref, p_ref, r_ref):
    x = x_ref[...]
    p_ref[...] = jnp.dot(x, wl_ref[...], preferred_element_type=jnp.float32) + c_ref[...]
    r_ref[...] = jnp.dot(x, wr_ref[...], preferred_element_type=jnp.float32) + b_ref[...]


def _proj(x, wl_aug, wr, c_aug, b):
    n, d = x.shape
    h = wr.shape[1]
    return pl.pallas_call(
        _proj_body,
        grid=(n // BM,),
        in_specs=[
            pl.BlockSpec((BM, d), lambda i: (i, 0)),
            pl.BlockSpec((d, W1), lambda i: (0, 0)),
            pl.BlockSpec((d, h), lambda i: (0, 0)),
            pl.BlockSpec((1, W1), lambda i: (0, 0)),
            pl.BlockSpec((1, h), lambda i: (0, 0)),
        ],
        out_specs=[
            pl.BlockSpec((BM, W1), lambda i: (i, 0)),
            pl.BlockSpec((BM, h), lambda i: (i, 0)),
        ],
        out_shape=[
            jax.ShapeDtypeStruct((n, W1), jnp.float32),
            jax.ShapeDtypeStruct((n, h), jnp.float32),
        ],
    )(x, wl_aug, wr, c_aug, b.reshape(1, h))


def _mid_body(agg_ref, r1_ref, wl_ref, wr_ref, b_ref, p_ref, rc_ref):
    a = agg_ref[:, :W1] + agg_ref[:, COFF:COFF + W1]
    cnt = jnp.maximum(a[:, D_HID:D_HID + 1], 1.0)
    h1 = jnp.maximum(a[:, :D_HID] / cnt + r1_ref[...], 0.0)
    p_ref[...] = jnp.dot(h1, wl_ref[...], preferred_element_type=jnp.float32)
    zr = jnp.dot(h1, wr_ref[...], preferred_element_type=jnp.float32) + b_ref[...]
    lane = lax.broadcasted_iota(jnp.int32, zr.shape, 1)
    rc_ref[...] = jnp.where(lane == D_HID, cnt, zr)


def _mid(agg, r1, wr_aug, b_aug, wl):
    n, h = r1.shape
    return pl.pallas_call(
        _mid_body,
        grid=(n // BM,),
        in_specs=[
            pl.BlockSpec((BM, 128), lambda i: (i, 0)),
            pl.BlockSpec((BM, h), lambda i: (i, 0)),
            pl.BlockSpec((h, h), lambda i: (0, 0)),
            pl.BlockSpec((h, W1), lambda i: (0, 0)),
            pl.BlockSpec((1, W1), lambda i: (0, 0)),
        ],
        out_specs=[
            pl.BlockSpec((BM, h), lambda i: (i, 0)),
            pl.BlockSpec((BM, W1), lambda i: (i, 0)),
        ],
        out_shape=[
            jax.ShapeDtypeStruct((n, h), jnp.float32),
            jax.ShapeDtypeStruct((n, W1), jnp.float32),
        ],
    )(agg, r1, wl, wr_aug, b_aug)


def _fin_body(agg_ref, rc_ref, wlin_ref, blin_ref, o_ref):
    a = agg_ref[:, :W2] + agg_ref[:, COFF:COFF + W2]
    h2 = a / rc_ref[:, D_HID:D_HID + 1] + rc_ref[:, :D_HID]
    z = jnp.dot(h2, wlin_ref[...], preferred_element_type=jnp.float32) + blin_ref[...]
    o_ref[...] = jax.nn.sigmoid(z)


def _fin(agg, rc2, wlin, blin):
    n = rc2.shape[0]
    h = D_HID
    return pl.pallas_call(
        _fin_body,
        grid=(n // BM,),
        in_specs=[
            pl.BlockSpec((BM, 128), lambda i: (i, 0)),
            pl.BlockSpec((BM, W1), lambda i: (i, 0)),
            pl.BlockSpec((h, 1), lambda i: (0, 0)),
            pl.BlockSpec((1, 1), lambda i: (0, 0)),
        ],
        out_specs=pl.BlockSpec((BM, 1), lambda i: (i, 0)),
        out_shape=jax.ShapeDtypeStruct((n, 1), jnp.float32),
    )(agg, rc2, wlin, blin.reshape(1, 1))


def _make_sc_pass(width):
    out_type = jax.ShapeDtypeStruct((NP, 128), jnp.float32)
    scratch = [
        pltpu.VMEM((TPW, CH), jnp.int32),
        pltpu.VMEM((TPW, CH), jnp.int32),
        pltpu.VMEM((1, CH), jnp.int32),
        pltpu.VMEM((1, CH), jnp.int32),
        pltpu.VMEM((NBUF, CH, width), jnp.float32),
        pltpu.VMEM_SHARED((NP, width), jnp.float32),
    ] + [pltpu.SemaphoreType.DMA] * (2 * NBUF)
    mesh = plsc.VectorSubcoreMesh(core_axis_name="c", subcore_axis_name="s")

    def body(p_hbm, ei_hbm, z_hbm, agg_out,
             sbuf, dbuf, sext, dext, rows, acc, *sems):
        gsem = sems[:NBUF]
        ssem = sems[NBUF:]
        cid = lax.axis_index("c")
        sid = lax.axis_index("s")
        wid = sid * NC + cid
        base = sid * RPS

        def g_start(t, b):
            pltpu.async_copy(p_hbm.at[sbuf.at[t]], rows.at[b], gsem[b])

        def g_wait(b):
            pltpu.make_async_copy(p_hbm.at[sbuf.at[0]], rows.at[b], gsem[b]).wait()

        def s_start(t, b):
            pltpu.async_copy(rows.at[b], acc.at[dbuf.at[t]], ssem[b], add=True)

        def s_wait(b):
            pltpu.make_async_copy(rows.at[b], acc.at[dbuf.at[0]], ssem[b]).wait()

        pltpu.sync_copy(z_hbm.at[pl.ds(base, RPS)], acc.at[pl.ds(base, RPS)])
        pltpu.sync_copy(ei_hbm.at[0, pl.ds(wid * TPW, TPW)], sbuf)
        pltpu.sync_copy(ei_hbm.at[1, pl.ds(wid * TPW, TPW)], dbuf)

        @pl.when(wid < NEXTRA)
        def _():
            pltpu.sync_copy(ei_hbm.at[0, pl.ds(NW * TPW + wid, 1)], sext)
            pltpu.sync_copy(ei_hbm.at[1, pl.ds(NW * TPW + wid, 1)], dext)

        plsc.subcore_barrier()

        for b in range(NBUF):
            g_start(b, b)

        nfull = TPW // NBUF

        def round_(u, carry):
            for b in range(NBUF):
                g_wait(b)
                s_start(u * NBUF + b, b)
            for b in range(NBUF):
                s_wait(b)
                t2 = (u + 1) * NBUF + b

                @pl.when(t2 < TPW)
                def _():
                    g_start(t2, b)

            return carry

        lax.fori_loop(0, nfull, round_, 0)

        for b in range(TPW - nfull * NBUF):
            g_wait(b)
            s_start(nfull * NBUF + b, b)
            s_wait(b)

        @pl.when(wid < NEXTRA)
        def _():
            pltpu.async_copy(p_hbm.at[sext.at[0]], rows.at[0], gsem[0])
            g_wait(0)
            pltpu.async_copy(rows.at[0], acc.at[dext.at[0]], ssem[0], add=True)
            s_wait(0)

        plsc.subcore_barrier()

        pltpu.sync_copy(acc.at[pl.ds(base, RPS)],
                        agg_out.at[pl.ds(base, RPS), pl.ds(cid * COFF, width)])

    return pl.kernel(body, out_type=out_type, mesh=mesh, scratch_types=scratch,
                     compiler_params=pltpu.CompilerParams(use_tc_tiling_on_sc=False))


_sc_pass40 = _make_sc_pass(W1)
_sc_pass32 = _make_sc_pass(W2)


def kernel(x, edge_index, W1l, W1r, b1, W2l, W2r, b2, Wlin, blin):
    ei3 = edge_index.astype(jnp.int32).reshape(2, NCHUNK, CH)
    wl_aug = jnp.pad(W1l, ((0, 0), (0, W1 - D_HID)))
    c_aug = jnp.zeros((1, W1), jnp.float32).at[0, D_HID].set(1.0)
    z40 = jnp.zeros((NP, W1), jnp.float32)
    z32 = jnp.zeros((NP, W2), jnp.float32)

    w2r_aug = jnp.pad(W2r, ((0, 0), (0, W1 - D_HID)))
    b2_aug = jnp.pad(b2, (0, W1 - D_HID)).reshape(1, W1)

    p1, r1 = _proj(x, wl_aug, W1r, c_aug, b1)
    agg1 = _sc_pass40(p1, ei3, z40)
    p2, rc2 = _mid(agg1, r1, w2r_aug, b2_aug, W2l)
    agg2 = _sc_pass32(p2, ei3, z32)
    outp = _fin(agg2, rc2, Wlin, blin)
    return {"product_order": outp}


# speedup vs baseline: 1.0283x; 1.0283x over previous
import jax
import jax.numpy as jnp
from jax import lax
from jax.experimental import pallas as pl
from jax.experimental.pallas import tpu as pltpu
from jax.experimental.pallas import tpu_sc as plsc

N_NODES = 10000
N_EDGES = 320000
D_IN = 128
D_HID = 32
W1 = 40
W2 = 32
COFF = 64

NC, NS = 2, 16
NW = NC * NS
CH = 128
NCHUNK = N_EDGES // CH
TPW = NCHUNK // NW
NEXTRA = NCHUNK - TPW * NW
NBUF = 13
NP = 10240
RPS = NP // NS
BM = 5000


def _proj_body(x_ref, wl_ref, wr_ref, c_ref, b_ref, p_ref, r_ref):
    x = x_ref[...]
    p_ref[...] = jnp.dot(x, wl_ref[...], preferred_element_type=jnp.float32) + c_ref[...]
    r_ref[...] = jnp.dot(x, wr_ref[...], preferred_element_type=jnp.float32) + b_ref[...]


def _proj(x, wl_aug, wr, c_aug, b):
    n, d = x.shape
    h = wr.shape[1]
    return pl.pallas_call(
        _proj_body,
        grid=(n // BM,),
        in_specs=[
            pl.BlockSpec((BM, d), lambda i: (i, 0)),
            pl.BlockSpec((d, W1), lambda i: (0, 0)),
            pl.BlockSpec((d, h), lambda i: (0, 0)),
            pl.BlockSpec((1, W1), lambda i: (0, 0)),
            pl.BlockSpec((1, h), lambda i: (0, 0)),
        ],
        out_specs=[
            pl.BlockSpec((BM, W1), lambda i: (i, 0)),
            pl.BlockSpec((BM, h), lambda i: (i, 0)),
        ],
        out_shape=[
            jax.ShapeDtypeStruct((n, W1), jnp.float32),
            jax.ShapeDtypeStruct((n, h), jnp.float32),
        ],
    )(x, wl_aug, wr, c_aug, b.reshape(1, h))


def _mid_body(agg_ref, r1_ref, wl_ref, wr_ref, b_ref, p_ref, rc_ref):
    a = agg_ref[:, :W1] + agg_ref[:, COFF:COFF + W1]
    cnt = jnp.maximum(a[:, D_HID:D_HID + 1], 1.0)
    h1 = jnp.maximum(a[:, :D_HID] / cnt + r1_ref[...], 0.0)
    p_ref[...] = jnp.dot(h1, wl_ref[...], preferred_element_type=jnp.float32)
    zr = jnp.dot(h1, wr_ref[...], preferred_element_type=jnp.float32) + b_ref[...]
    lane = lax.broadcasted_iota(jnp.int32, zr.shape, 1)
    rc_ref[...] = jnp.where(lane == D_HID, cnt, zr)


def _mid(agg, r1, wr_aug, b_aug, wl):
    n, h = r1.shape
    return pl.pallas_call(
        _mid_body,
        grid=(n // BM,),
        in_specs=[
            pl.BlockSpec((BM, 128), lambda i: (i, 0)),
            pl.BlockSpec((BM, h), lambda i: (i, 0)),
            pl.BlockSpec((h, h), lambda i: (0, 0)),
            pl.BlockSpec((h, W1), lambda i: (0, 0)),
            pl.BlockSpec((1, W1), lambda i: (0, 0)),
        ],
        out_specs=[
            pl.BlockSpec((BM, h), lambda i: (i, 0)),
            pl.BlockSpec((BM, W1), lambda i: (i, 0)),
        ],
        out_shape=[
            jax.ShapeDtypeStruct((n, h), jnp.float32),
            jax.ShapeDtypeStruct((n, W1), jnp.float32),
        ],
    )(agg, r1, wl, wr_aug, b_aug)


def _fin_body(agg_ref, rc_ref, wlin_ref, blin_ref, o_ref):
    a = agg_ref[:, :W2] + agg_ref[:, COFF:COFF + W2]
    h2 = a / rc_ref[:, D_HID:D_HID + 1] + rc_ref[:, :D_HID]
    z = jnp.dot(h2, wlin_ref[...], preferred_element_type=jnp.float32) + blin_ref[...]
    o_ref[...] = jax.nn.sigmoid(z)


def _fin(agg, rc2, wlin, blin):
    n = rc2.shape[0]
    h = D_HID
    return pl.pallas_call(
        _fin_body,
        grid=(n // BM,),
        in_specs=[
            pl.BlockSpec((BM, 128), lambda i: (i, 0)),
            pl.BlockSpec((BM, W1), lambda i: (i, 0)),
            pl.BlockSpec((h, 1), lambda i: (0, 0)),
            pl.BlockSpec((1, 1), lambda i: (0, 0)),
        ],
        out_specs=pl.BlockSpec((BM, 1), lambda i: (i, 0)),
        out_shape=jax.ShapeDtypeStruct((n, 1), jnp.float32),
    )(agg, rc2, wlin, blin.reshape(1, 1))


def _make_sc_pass(width):
    out_type = jax.ShapeDtypeStruct((NP, 128), jnp.float32)
    scratch = [
        pltpu.VMEM((TPW, CH), jnp.int32),
        pltpu.VMEM((TPW, CH), jnp.int32),
        pltpu.VMEM((1, CH), jnp.int32),
        pltpu.VMEM((1, CH), jnp.int32),
        pltpu.VMEM((NBUF, CH, width), jnp.float32),
        pltpu.VMEM_SHARED((NP, width), jnp.float32),
    ] + [pltpu.SemaphoreType.DMA] * (2 * NBUF)
    mesh = plsc.VectorSubcoreMesh(core_axis_name="c", subcore_axis_name="s")

    def body(p_hbm, ei_hbm, z_hbm, agg_out,
             sbuf, dbuf, sext, dext, rows, acc, *sems):
        gsem = sems[:NBUF]
        ssem = sems[NBUF:]
        cid = lax.axis_index("c")
        sid = lax.axis_index("s")
        wid = sid * NC + cid
        base = sid * RPS

        def g_start(t, b):
            pltpu.async_copy(p_hbm.at[sbuf.at[t]], rows.at[b], gsem[b])

        def g_wait(b):
            pltpu.make_async_copy(p_hbm.at[sbuf.at[0]], rows.at[b], gsem[b]).wait()

        def s_start(t, b):
            pltpu.async_copy(rows.at[b], acc.at[dbuf.at[t]], ssem[b], add=True)

        def s_wait(b):
            pltpu.make_async_copy(rows.at[b], acc.at[dbuf.at[0]], ssem[b]).wait()

        pltpu.sync_copy(z_hbm.at[pl.ds(base, RPS)], acc.at[pl.ds(base, RPS)])
        pltpu.sync_copy(ei_hbm.at[0, pl.ds(wid * TPW, TPW)], sbuf)
        pltpu.sync_copy(ei_hbm.at[1, pl.ds(wid * TPW, TPW)], dbuf)

        @pl.when(wid < NEXTRA)
        def _():
            pltpu.sync_copy(ei_hbm.at[0, pl.ds(NW * TPW + wid, 1)], sext)
            pltpu.sync_copy(ei_hbm.at[1, pl.ds(NW * TPW + wid, 1)], dext)

        plsc.subcore_barrier()

        for b in range(NBUF):
            g_start(b, b)

        nfull = TPW // NBUF

        def round_(u, carry):
            for b in range(NBUF):
                g_wait(b)
                s_start(u * NBUF + b, b)
            for b in range(NBUF):
                s_wait(b)
                t2 = (u + 1) * NBUF + b

                @pl.when(t2 < TPW)
                def _():
                    g_start(t2, b)

            return carry

        lax.fori_loop(0, nfull, round_, 0)

        for b in range(TPW - nfull * NBUF):
            g_wait(b)
            s_start(nfull * NBUF + b, b)
            s_wait(b)

        @pl.when(wid < NEXTRA)
        def _():
            pltpu.async_copy(p_hbm.at[sext.at[0]], rows.at[0], gsem[0])
            g_wait(0)
            pltpu.async_copy(rows.at[0], acc.at[dext.at[0]], ssem[0], add=True)
            s_wait(0)

        plsc.subcore_barrier()

        pltpu.sync_copy(acc.at[pl.ds(base, RPS)],
                        agg_out.at[pl.ds(base, RPS), pl.ds(cid * COFF, width)])

    return pl.kernel(body, out_type=out_type, mesh=mesh, scratch_types=scratch,
                     compiler_params=pltpu.CompilerParams(use_tc_tiling_on_sc=False))


_sc_pass40 = _make_sc_pass(W1)
_sc_pass32 = _make_sc_pass(W2)


def kernel(x, edge_index, W1l, W1r, b1, W2l, W2r, b2, Wlin, blin):
    ei3 = edge_index.astype(jnp.int32).reshape(2, NCHUNK, CH)
    wl_aug = jnp.pad(W1l, ((0, 0), (0, W1 - D_HID)))
    c_aug = jnp.zeros((1, W1), jnp.float32).at[0, D_HID].set(1.0)
    z40 = jnp.zeros((NP, W1), jnp.float32)
    z32 = jnp.zeros((NP, W2), jnp.float32)

    w2r_aug = jnp.pad(W2r, ((0, 0), (0, W1 - D_HID)))
    b2_aug = jnp.pad(b2, (0, W1 - D_HID)).reshape(1, W1)

    p1, r1 = _proj(x, wl_aug, W1r, c_aug, b1)
    agg1 = _sc_pass40(p1, ei3, z40)
    p2, rc2 = _mid(agg1, r1, w2r_aug, b2_aug, W2l)
    agg2 = _sc_pass32(p2, ei3, z32)
    outp = _fin(agg2, rc2, Wlin, blin)
    return {"product_order": outp}
